# baseline (speedup 1.0000x reference)
import functools

import jax
import jax.numpy as jnp
from jax import lax
from jax.experimental import pallas as pl
from jax.experimental.pallas import tpu as pltpu
from jax.experimental.pallas import tpu_sc as plsc

N = 10000
E = 320000
NREL = 4
DH = 64
HALF = 32
ROWS = NREL * N
NS = 16
CH = 100
EPW = E // NS
IT_MAIN = EPW // CH
EPC = E // (2 * NS)
IT_CNT = EPC // CH
NS_IO = 8
RPS = ROWS // NS_IO


BN = 2000
NB = N // BN
D_IN = 128


def _dot(a, b):
    return lax.dot_general(a, b, (((1,), (0,)), ((), ())),
                           preferred_element_type=jnp.float32)


def _tc_a_body(x_ref, w_ref, root_ref, b_ref, src_ref, dst_ref, et_ref,
               ha_ref, hb_ref, base_ref, g_ref, s_ref):
    x = x_ref[...]
    for r in range(NREL):
        h = _dot(x, w_ref[r])
        ha_ref[r] = h[:, :HALF]
        hb_ref[r] = h[:, HALF:]
    base_ref[...] = _dot(x, root_ref[...]) + b_ref[...]

    @pl.when(pl.program_id(0) == 0)
    def _():
        et = et_ref[...]
        g_ref[...] = et * N + src_ref[...]
        s_ref[...] = et * N + dst_ref[...]


def _tc_a(x, W0, root0, b0, src2, dst2, et2):
    full_e = pl.BlockSpec((2500, 128), lambda i: (0, 0))
    return pl.pallas_call(
        _tc_a_body,
        grid=(NB,),
        in_specs=[
            pl.BlockSpec((BN, D_IN), lambda i: (i, 0)),
            pl.BlockSpec((NREL, D_IN, DH), lambda i: (0, 0, 0)),
            pl.BlockSpec((D_IN, DH), lambda i: (0, 0)),
            pl.BlockSpec((1, DH), lambda i: (0, 0)),
            full_e, full_e, full_e,
        ],
        out_specs=[
            pl.BlockSpec((NREL, BN, HALF), lambda i: (0, i, 0)),
            pl.BlockSpec((NREL, BN, HALF), lambda i: (0, i, 0)),
            pl.BlockSpec((BN, DH), lambda i: (i, 0)),
            full_e, full_e,
        ],
        out_shape=[
            jax.ShapeDtypeStruct((NREL, N, HALF), jnp.float32),
            jax.ShapeDtypeStruct((NREL, N, HALF), jnp.float32),
            jax.ShapeDtypeStruct((N, DH), jnp.float32),
            jax.ShapeDtypeStruct(src2.shape, jnp.int32),
            jax.ShapeDtypeStruct(src2.shape, jnp.int32),
        ],
    )(x, W0, root0, b0.reshape(1, DH), src2, dst2, et2)


def _combine(agg_ref, cnt_ref, base_ref):
    ct = cnt_ref[:, :NREL] + cnt_ref[:, NREL:]
    recip = 1.0 / jnp.maximum(ct, 1.0)
    acc = base_ref[...]
    for r in range(NREL):
        m = jnp.concatenate([agg_ref[0, r], agg_ref[1, r]], axis=-1)
        acc = acc + m * recip[:, r:r + 1]
    return jnp.maximum(acc, 0.0)


def _tc_b_body(agg_ref, cnt_ref, base_ref, w_ref, root_ref, b_ref,
               x1_ref, ha_ref, hb_ref, base1_ref):
    x1 = _combine(agg_ref, cnt_ref, base_ref)
    x1_ref[...] = x1
    for r in range(NREL):
        h = _dot(x1, w_ref[r])
        ha_ref[r] = h[:, :HALF]
        hb_ref[r] = h[:, HALF:]
    base1_ref[...] = _dot(x1, root_ref[...]) + b_ref[...]


def _tc_b(aggfull, cntc, base0, W1, root1, b1):
    return pl.pallas_call(
        _tc_b_body,
        grid=(NB,),
        in_specs=[
            pl.BlockSpec((2, NREL, BN, HALF), lambda i: (0, 0, i, 0)),
            pl.BlockSpec((BN, 2 * NREL), lambda i: (i, 0)),
            pl.BlockSpec((BN, DH), lambda i: (i, 0)),
            pl.BlockSpec((NREL, DH, DH), lambda i: (0, 0, 0)),
            pl.BlockSpec((DH, DH), lambda i: (0, 0)),
            pl.BlockSpec((1, DH), lambda i: (0, 0)),
        ],
        out_specs=[
            pl.BlockSpec((BN, DH), lambda i: (i, 0)),
            pl.BlockSpec((NREL, BN, HALF), lambda i: (0, i, 0)),
            pl.BlockSpec((NREL, BN, HALF), lambda i: (0, i, 0)),
            pl.BlockSpec((BN, DH), lambda i: (i, 0)),
        ],
        out_shape=[
            jax.ShapeDtypeStruct((N, DH), jnp.float32),
            jax.ShapeDtypeStruct((NREL, N, HALF), jnp.float32),
            jax.ShapeDtypeStruct((NREL, N, HALF), jnp.float32),
            jax.ShapeDtypeStruct((N, DH), jnp.float32),
        ],
    )(aggfull, cntc, base0, W1, root1, b1.reshape(1, DH))


def _tc_c_body(agg_ref, cnt_ref, base_ref, x1_ref, batch_ref,
               fc1w_ref, fc1b_ref, fc15w_ref, fc15b_ref, fc2w_ref, fc2b_ref,
               out_ref, gs_ref, gc_ref):
    i = pl.program_id(0)

    @pl.when(i == 0)
    def _():
        gs_ref[...] = jnp.zeros((16, 2 * DH), jnp.float32)
        gc_ref[...] = jnp.zeros((16, 2 * DH), jnp.float32)

    x2 = _combine(agg_ref, cnt_ref, base_ref)
    xc = jnp.concatenate([x1_ref[...], x2], axis=-1)
    bcol = batch_ref[...]
    gids = lax.broadcasted_iota(jnp.int32, (BN, 16), 1)
    oh = (bcol == gids).astype(jnp.float32)
    gs_ref[...] += lax.dot_general(oh, xc, (((0,), (0,)), ((), ())),
                                   preferred_element_type=jnp.float32)
    gc_ref[...] += lax.dot_general(oh, jnp.ones((BN, 2 * DH), jnp.float32),
                                   (((0,), (0,)), ((), ())),
                                   preferred_element_type=jnp.float32)

    @pl.when(i == NB - 1)
    def _():
        g = gs_ref[...] / jnp.maximum(gc_ref[...], 1.0)
        h1 = jnp.maximum(_dot(g, fc1w_ref[...]) + fc1b_ref[...], 0.0)
        hm = lax.dot_general(jnp.full((1, 16), 1.0 / 16.0, jnp.float32), h1,
                             (((1,), (0,)), ((), ())),
                             preferred_element_type=jnp.float32)
        h2 = jnp.maximum(_dot(hm, fc15w_ref[...]) + fc15b_ref[...], 0.0)
        logits = _dot(h2, fc2w_ref[...]) + fc2b_ref[...]
        m = jnp.max(logits, axis=1, keepdims=True)
        ssum = jnp.sum(jnp.exp(logits - m), axis=1, keepdims=True)
        out_ref[...] = logits - m - jnp.log(ssum)


def _tc_c(aggfull, cntc, base1, x1, batchT, fc1_w, fc1_b, fc15_w, fc15_b,
          fc2_w, fc2_b):
    return pl.pallas_call(
        _tc_c_body,
        grid=(NB,),
        in_specs=[
            pl.BlockSpec((2, NREL, BN, HALF), lambda i: (0, 0, i, 0)),
            pl.BlockSpec((BN, 2 * NREL), lambda i: (i, 0)),
            pl.BlockSpec((BN, DH), lambda i: (i, 0)),
            pl.BlockSpec((BN, DH), lambda i: (i, 0)),
            pl.BlockSpec((BN, 1), lambda i: (i, 0)),
            pl.BlockSpec((2 * DH, 128), lambda i: (0, 0)),
            pl.BlockSpec((1, 128), lambda i: (0, 0)),
            pl.BlockSpec((128, 64), lambda i: (0, 0)),
            pl.BlockSpec((1, 64), lambda i: (0, 0)),
            pl.BlockSpec((64, 8), lambda i: (0, 0)),
            pl.BlockSpec((1, 8), lambda i: (0, 0)),
        ],
        out_specs=[pl.BlockSpec((1, 8), lambda i: (0, 0))],
        out_shape=[jax.ShapeDtypeStruct((1, 8), jnp.float32)],
        scratch_shapes=[
            pltpu.VMEM((16, 2 * DH), jnp.float32),
            pltpu.VMEM((16, 2 * DH), jnp.float32),
        ],
    )(aggfull, cntc, base1, x1, batchT,
      fc1_w, fc1_b.reshape(1, -1), fc15_w, fc15_b.reshape(1, -1),
      fc2_w, fc2_b.reshape(1, -1))


def _make_sc_agg():
    mesh = plsc.VectorSubcoreMesh(core_axis_name="c", subcore_axis_name="s")
    out_type = [jax.ShapeDtypeStruct((2, NREL, N, HALF), jnp.float32)]
    IT2 = IT_MAIN // 2
    scratch = [
        pltpu.VMEM_SHARED((ROWS, HALF), jnp.float32),
        pltpu.VMEM((IT2, CH), jnp.int32),
        pltpu.VMEM((IT2, CH), jnp.int32),
        pltpu.VMEM((CH, HALF), jnp.float32),
        pltpu.VMEM((CH, HALF), jnp.float32),
        pltpu.VMEM((CH, HALF), jnp.float32),
        pltpu.VMEM((CH, HALF), jnp.float32),
        pltpu.SemaphoreType.DMA,
        pltpu.SemaphoreType.DMA,
        pltpu.SemaphoreType.DMA,
        pltpu.SemaphoreType.DMA,
    ]

    def body(ha, hb, g3, s3, z32, agg_out, ACC, gv, sv, rowsa, rowsb,
             rowsc, rowsd, sema, semb, semc, semd):
        c = lax.axis_index("c")
        s = lax.axis_index("s")

        @pl.when(s < NS_IO)
        def _():
            pltpu.sync_copy(z32, ACC.at[pl.ds(s * RPS, RPS)])
        plsc.subcore_barrier()

        def run(table):
            bufs = ((rowsa, sema), (rowsb, semb), (rowsc, semc), (rowsd, semd))

            for p in range(2):
                pltpu.sync_copy(g3.at[s].at[pl.ds(p * IT2, IT2)], gv)
                pltpu.sync_copy(s3.at[s].at[pl.ds(p * IT2, IT2)], sv)

                def it(j, carry):
                    i0 = 4 * j
                    cps = [pltpu.async_copy(table.at[gv.at[i0 + k]], buf, sem)
                           for k, (buf, sem) in enumerate(bufs)]
                    for k, (buf, _) in enumerate(bufs):
                        cps[k].wait()
                        pltpu.sync_copy(buf, ACC.at[sv.at[i0 + k]], add=True)
                    return carry
                lax.fori_loop(0, IT2 // 4, it, 0)

        @pl.when(c == 0)
        def _():
            run(ha)

        @pl.when(c == 1)
        def _():
            run(hb)

        plsc.subcore_barrier()

        @pl.when(s < NS_IO)
        def _():
            pltpu.sync_copy(
                ACC.at[pl.ds(s * RPS, RPS)],
                agg_out.at[c].at[s // 2].at[pl.ds((s % 2) * RPS, RPS)])

    return pl.kernel(body, out_type=out_type, mesh=mesh,
                     scratch_types=scratch,
                     compiler_params=pltpu.CompilerParams(
                         use_tc_tiling_on_sc=False))


def _make_sc_cnt():
    mesh = plsc.VectorSubcoreMesh(core_axis_name="c", subcore_axis_name="s")
    out_type = [jax.ShapeDtypeStruct((2, ROWS, 16), jnp.float32)]
    scratch = [
        pltpu.VMEM_SHARED((ROWS, 16), jnp.float32),
        pltpu.VMEM((IT_CNT, CH), jnp.int32),
        pltpu.VMEM((CH, 16), jnp.float32),
    ]

    def body(s3, z16, e0, cnt_out, CNT, scv, e0v):
        c = lax.axis_index("c")
        s = lax.axis_index("s")

        @pl.when(s < NS_IO)
        def _():
            pltpu.sync_copy(z16, CNT.at[pl.ds(s * RPS, RPS)])
        w = c * NS + s
        pltpu.sync_copy(
            s3.at[w // 2].at[pl.ds((w % 2) * IT_CNT, IT_CNT)], scv)
        pltpu.sync_copy(e0, e0v)
        plsc.subcore_barrier()

        def itc(i, carry):
            pltpu.sync_copy(e0v, CNT.at[scv.at[i]], add=True)
            return carry
        lax.fori_loop(0, IT_CNT, itc, 0)

        plsc.subcore_barrier()

        @pl.when(s < NS_IO)
        def _():
            pltpu.sync_copy(CNT.at[pl.ds(s * RPS, RPS)],
                            cnt_out.at[c, pl.ds(s * RPS, RPS)])

    return pl.kernel(body, out_type=out_type, mesh=mesh,
                     scratch_types=scratch,
                     compiler_params=pltpu.CompilerParams(
                         use_tc_tiling_on_sc=False))


_sc_agg = _make_sc_agg()
_sc_cnt = _make_sc_cnt()


def kernel(x, edge_index, edge_attr, batch, W0, root0, b0, W1, root1, b1,
           fc1_w, fc1_b, fc15_w, fc15_b, fc2_w, fc2_b):
    src2 = edge_index[0].reshape(2500, 128)
    dst2 = edge_index[1].reshape(2500, 128)
    et2 = edge_attr.reshape(2500, 128)

    Ha0, Hb0, base0, gidx, sidx = _tc_a(x, W0, root0, b0, src2, dst2, et2)
    g3 = gidx.reshape(NS, IT_MAIN, CH)
    s3 = sidx.reshape(NS, IT_MAIN, CH)

    ha0 = Ha0.reshape(ROWS, HALF)
    hb0 = Hb0.reshape(ROWS, HALF)
    z32 = jnp.zeros((RPS, HALF), jnp.float32)
    z16 = jnp.zeros((RPS, 16), jnp.float32)
    e0 = jnp.zeros((CH, 16), jnp.float32).at[:, 0].set(1.0)

    (cnt,) = _sc_cnt(s3, z16, e0)
    (agg0,) = _sc_agg(ha0, hb0, g3, s3, z32)
    cntc = cnt[:, :, 0].reshape(2, NREL, N).transpose(2, 0, 1).reshape(
        N, 2 * NREL)

    x1, Ha1, Hb1, base1 = _tc_b(agg0, cntc, base0, W1, root1, b1)
    ha1 = Ha1.reshape(ROWS, HALF)
    hb1 = Hb1.reshape(ROWS, HALF)

    (agg1,) = _sc_agg(ha1, hb1, g3, s3, z32)

    (out,) = _tc_c(agg1, cntc, base1, x1, batch.reshape(N, 1),
                   fc1_w, fc1_b, fc15_w, fc15_b, fc2_w, fc2_b)
    return out.reshape(8)


# speedup vs baseline: 1.1527x; 1.1527x over previous
import functools

import jax
import jax.numpy as jnp
from jax import lax
from jax.experimental import pallas as pl
from jax.experimental.pallas import tpu as pltpu
from jax.experimental.pallas import tpu_sc as plsc

N = 10000
E = 320000
NREL = 4
DH = 64
HALF = 32
ROWS = NREL * N
NP4 = N // 4
NS = 16
CH = 100
EPW = E // NS
IT_MAIN = EPW // CH
EPC = E // (2 * NS)
IT_CNT = EPC // CH
NS_IO = 8
RPS = ROWS // NS_IO
D_IN = 128


def _dot(a, b):
    return lax.dot_general(a, b, (((1,), (0,)), ((), ())),
                           preferred_element_type=jnp.float32)


def _tc_a_body(x4_ref, wa_ref, wb_ref, root4_ref, b4_ref,
               src_ref, dst_ref, et_ref,
               ha_ref, hb_ref, base_ref, g_ref, s_ref):
    x4 = x4_ref[...]
    for r in range(NREL):
        ha_ref[r] = _dot(x4, wa_ref[r])
        hb_ref[r] = _dot(x4, wb_ref[r])
    base_ref[...] = _dot(x4, root4_ref[...]) + b4_ref[...]
    et = et_ref[...]
    g_ref[...] = et * N + src_ref[...]
    s_ref[...] = et * N + dst_ref[...]


def _tc_a(x4, Wa, Wb, root4, b4, src2, dst2, et2):
    return pl.pallas_call(
        _tc_a_body,
        out_shape=[
            jax.ShapeDtypeStruct((NREL, NP4, 128), jnp.float32),
            jax.ShapeDtypeStruct((NREL, NP4, 128), jnp.float32),
            jax.ShapeDtypeStruct((NP4, 4 * DH), jnp.float32),
            jax.ShapeDtypeStruct(src2.shape, jnp.int32),
            jax.ShapeDtypeStruct(src2.shape, jnp.int32),
        ],
    )(x4, Wa, Wb, root4, b4, src2, dst2, et2)


def _combine(agg_ref, cntp_ref, base_ref, e_ref):
    cntp = cntp_ref[...]
    c0 = jnp.concatenate([cntp[:, 0:4], cntp[:, 8:12],
                          cntp[:, 16:20], cntp[:, 24:28]], axis=1)
    c1 = jnp.concatenate([cntp[:, 4:8], cntp[:, 12:16],
                          cntp[:, 20:24], cntp[:, 28:32]], axis=1)
    recip16 = 1.0 / jnp.maximum(c0 + c1, 1.0)
    acc = base_ref[...]
    for r in range(NREL):
        rp = _dot(recip16, e_ref[r])
        m0 = agg_ref[0, r] * rp
        m1 = agg_ref[1, r] * rp
        contrib = jnp.concatenate(
            [m0[:, 0:32], m1[:, 0:32], m0[:, 32:64], m1[:, 32:64],
             m0[:, 64:96], m1[:, 64:96], m0[:, 96:128], m1[:, 96:128]],
            axis=1)
        acc = acc + contrib
    return jnp.maximum(acc, 0.0)


def _tc_b_body(agg_ref, cntp_ref, base_ref, e_ref, wa_ref, wb_ref,
               root4_ref, b4_ref, x1_ref, ha_ref, hb_ref, base1_ref):
    x1 = _combine(agg_ref, cntp_ref, base_ref, e_ref)
    x1_ref[...] = x1
    for r in range(NREL):
        ha_ref[r] = _dot(x1, wa_ref[r])
        hb_ref[r] = _dot(x1, wb_ref[r])
    base1_ref[...] = _dot(x1, root4_ref[...]) + b4_ref[...]


def _tc_b(aggp, cntp, base0, Emat, Wa, Wb, root4, b4):
    return pl.pallas_call(
        _tc_b_body,
        out_shape=[
            jax.ShapeDtypeStruct((NP4, 4 * DH), jnp.float32),
            jax.ShapeDtypeStruct((NREL, NP4, 128), jnp.float32),
            jax.ShapeDtypeStruct((NREL, NP4, 128), jnp.float32),
            jax.ShapeDtypeStruct((NP4, 4 * DH), jnp.float32),
        ],
    )(aggp, cntp, base0, Emat, Wa, Wb, root4, b4)


def _tc_c_body(agg_ref, cntp_ref, base_ref, e_ref, x1_ref, batch_ref,
               fc1w_ref, fc1b_ref, fc15w_ref, fc15b_ref, fc2w_ref, fc2b_ref,
               out_ref):
    x2 = _combine(agg_ref, cntp_ref, base_ref, e_ref)
    x1 = x1_ref[...]
    batchp = batch_ref[...]
    gs_parts = []
    gc = jnp.zeros((16, 8), jnp.float32)
    ones1 = jnp.ones((NP4, 8), jnp.float32)
    oh_list = []
    for k in range(4):
        gids = lax.broadcasted_iota(jnp.int32, (NP4, 16), 1)
        oh = (batchp[:, k:k + 1] == gids).astype(jnp.float32)
        oh_list.append(oh)
        gc = gc + lax.dot_general(oh, ones1, (((0,), (0,)), ((), ())),
                                  preferred_element_type=jnp.float32)
    s1 = sum(lax.dot_general(oh_list[k], x1[:, 64 * k:64 * k + 64],
                             (((0,), (0,)), ((), ())),
                             preferred_element_type=jnp.float32)
             for k in range(4))
    s2 = sum(lax.dot_general(oh_list[k], x2[:, 64 * k:64 * k + 64],
                             (((0,), (0,)), ((), ())),
                             preferred_element_type=jnp.float32)
             for k in range(4))
    gs = jnp.concatenate([s1, s2], axis=1)
    g = gs / jnp.maximum(gc[:, 0:1], 1.0)
    h1 = jnp.maximum(_dot(g, fc1w_ref[...]) + fc1b_ref[...], 0.0)
    hm = lax.dot_general(jnp.full((1, 16), 1.0 / 16.0, jnp.float32), h1,
                         (((1,), (0,)), ((), ())),
                         preferred_element_type=jnp.float32)
    h2 = jnp.maximum(_dot(hm, fc15w_ref[...]) + fc15b_ref[...], 0.0)
    logits = _dot(h2, fc2w_ref[...]) + fc2b_ref[...]
    m = jnp.max(logits, axis=1, keepdims=True)
    ssum = jnp.sum(jnp.exp(logits - m), axis=1, keepdims=True)
    out_ref[...] = logits - m - jnp.log(ssum)


def _tc_c(aggp, cntp, base1, Emat, x1, batchp, fc1_w, fc1_b, fc15_w, fc15_b,
          fc2_w, fc2_b):
    return pl.pallas_call(
        _tc_c_body,
        out_shape=[jax.ShapeDtypeStruct((1, 8), jnp.float32)],
    )(aggp, cntp, base1, Emat, x1, batchp,
      fc1_w, fc1_b.reshape(1, -1), fc15_w, fc15_b.reshape(1, -1),
      fc2_w, fc2_b.reshape(1, -1))


def _make_sc_agg():
    mesh = plsc.VectorSubcoreMesh(core_axis_name="c", subcore_axis_name="s")
    out_type = [jax.ShapeDtypeStruct((2, NREL, N, HALF), jnp.float32)]
    IT2 = IT_MAIN // 2
    scratch = [
        pltpu.VMEM_SHARED((ROWS, HALF), jnp.float32),
        pltpu.VMEM((IT2, CH), jnp.int32),
        pltpu.VMEM((IT2, CH), jnp.int32),
        pltpu.VMEM((CH, HALF), jnp.float32),
        pltpu.VMEM((CH, HALF), jnp.float32),
        pltpu.VMEM((CH, HALF), jnp.float32),
        pltpu.VMEM((CH, HALF), jnp.float32),
        pltpu.SemaphoreType.DMA,
        pltpu.SemaphoreType.DMA,
        pltpu.SemaphoreType.DMA,
        pltpu.SemaphoreType.DMA,
    ]

    def body(ha, hb, g3, s3, z32, agg_out, ACC, gv, sv, rowsa, rowsb,
             rowsc, rowsd, sema, semb, semc, semd):
        c = lax.axis_index("c")
        s = lax.axis_index("s")

        @pl.when(s < NS_IO)
        def _():
            pltpu.sync_copy(z32, ACC.at[pl.ds(s * RPS, RPS)])
        plsc.subcore_barrier()

        def run(table):
            bufs = ((rowsa, sema), (rowsb, semb), (rowsc, semc), (rowsd, semd))

            for p in range(2):
                pltpu.sync_copy(g3.at[s].at[pl.ds(p * IT2, IT2)], gv)
                pltpu.sync_copy(s3.at[s].at[pl.ds(p * IT2, IT2)], sv)

                def it(j, carry):
                    i0 = 4 * j
                    cps = [pltpu.async_copy(table.at[gv.at[i0 + k]], buf, sem)
                           for k, (buf, sem) in enumerate(bufs)]
                    for k, (buf, _) in enumerate(bufs):
                        cps[k].wait()
                        pltpu.sync_copy(buf, ACC.at[sv.at[i0 + k]], add=True)
                    return carry
                lax.fori_loop(0, IT2 // 4, it, 0)

        @pl.when(c == 0)
        def _():
            run(ha)

        @pl.when(c == 1)
        def _():
            run(hb)

        plsc.subcore_barrier()

        @pl.when(s < NS_IO)
        def _():
            pltpu.sync_copy(
                ACC.at[pl.ds(s * RPS, RPS)],
                agg_out.at[c].at[s // 2].at[pl.ds((s % 2) * RPS, RPS)])

    return pl.kernel(body, out_type=out_type, mesh=mesh,
                     scratch_types=scratch,
                     compiler_params=pltpu.CompilerParams(
                         use_tc_tiling_on_sc=False))


def _make_sc_cnt():
    mesh = plsc.VectorSubcoreMesh(core_axis_name="c", subcore_axis_name="s")
    out_type = [jax.ShapeDtypeStruct((2, ROWS, 16), jnp.float32)]
    scratch = [
        pltpu.VMEM_SHARED((ROWS, 16), jnp.float32),
        pltpu.VMEM((IT_CNT, CH), jnp.int32),
        pltpu.VMEM((CH, 16), jnp.float32),
    ]

    def body(s3, z16, e0, cnt_out, CNT, scv, e0v):
        c = lax.axis_index("c")
        s = lax.axis_index("s")

        @pl.when(s < NS_IO)
        def _():
            pltpu.sync_copy(z16, CNT.at[pl.ds(s * RPS, RPS)])
        w = c * NS + s
        pltpu.sync_copy(
            s3.at[w // 2].at[pl.ds((w % 2) * IT_CNT, IT_CNT)], scv)
        pltpu.sync_copy(e0, e0v)
        plsc.subcore_barrier()

        def itc(i, carry):
            pltpu.sync_copy(e0v, CNT.at[scv.at[i]], add=True)
            return carry
        lax.fori_loop(0, IT_CNT, itc, 0)

        plsc.subcore_barrier()

        @pl.when(s < NS_IO)
        def _():
            pltpu.sync_copy(CNT.at[pl.ds(s * RPS, RPS)],
                            cnt_out.at[c, pl.ds(s * RPS, RPS)])

    return pl.kernel(body, out_type=out_type, mesh=mesh,
                     scratch_types=scratch,
                     compiler_params=pltpu.CompilerParams(
                         use_tc_tiling_on_sc=False))


_sc_agg = _make_sc_agg()
_sc_cnt = _make_sc_cnt()


def _blockdiag4(w):
    return jax.scipy.linalg.block_diag(w, w, w, w)


def _packed_weights(W, root, b):
    Wa = jnp.stack([_blockdiag4(W[r][:, :HALF]) for r in range(NREL)])
    Wb = jnp.stack([_blockdiag4(W[r][:, HALF:]) for r in range(NREL)])
    root4 = _blockdiag4(root)
    b4 = jnp.tile(b, 4).reshape(1, 4 * DH)
    return Wa, Wb, root4, b4


def _expansion_consts():
    e = jnp.zeros((NREL, 16, 128), jnp.float32)
    for r in range(NREL):
        for k in range(4):
            e = e.at[r, 4 * k + r, 32 * k:32 * k + 32].set(1.0)
    return e


def kernel(x, edge_index, edge_attr, batch, W0, root0, b0, W1, root1, b1,
           fc1_w, fc1_b, fc15_w, fc15_b, fc2_w, fc2_b):
    src2 = edge_index[0].reshape(2500, 128)
    dst2 = edge_index[1].reshape(2500, 128)
    et2 = edge_attr.reshape(2500, 128)

    x4 = x.reshape(NP4, 4 * D_IN)
    Wa0, Wb0, root40, b40 = _packed_weights(W0, root0, b0)
    Wa1, Wb1, root41, b41 = _packed_weights(W1, root1, b1)
    Emat = _expansion_consts()

    Ha0, Hb0, base0, gidx, sidx = _tc_a(x4, Wa0, Wb0, root40, b40,
                                        src2, dst2, et2)
    g3 = gidx.reshape(NS, IT_MAIN, CH)
    s3 = sidx.reshape(NS, IT_MAIN, CH)

    ha0 = Ha0.reshape(ROWS, HALF)
    hb0 = Hb0.reshape(ROWS, HALF)
    z32 = jnp.zeros((RPS, HALF), jnp.float32)
    z16 = jnp.zeros((RPS, 16), jnp.float32)
    e0 = jnp.zeros((CH, 16), jnp.float32).at[:, 0].set(1.0)

    (cnt,) = _sc_cnt(s3, z16, e0)
    (agg0,) = _sc_agg(ha0, hb0, g3, s3, z32)
    cntp = cnt[:, :, 0].reshape(2, NREL, NP4, 4).transpose(2, 3, 0, 1)
    cntp = cntp.reshape(NP4, 32)

    agg0p = agg0.reshape(2, NREL, NP4, 128)
    x1, Ha1, Hb1, base1 = _tc_b(agg0p, cntp, base0, Emat, Wa1, Wb1,
                                root41, b41)
    ha1 = Ha1.reshape(ROWS, HALF)
    hb1 = Hb1.reshape(ROWS, HALF)

    (agg1,) = _sc_agg(ha1, hb1, g3, s3, z32)

    agg1p = agg1.reshape(2, NREL, NP4, 128)
    batchp = batch.reshape(NP4, 4)
    (out,) = _tc_c(agg1p, cntp, base1, Emat, x1, batchp,
                   fc1_w, fc1_b, fc15_w, fc15_b, fc2_w, fc2_b)
    return out.reshape(8)
